# Initial kernel scaffold; baseline (speedup 1.0000x reference)
import jax
import jax.numpy as jnp
from jax.experimental import pallas as pl


def kernel(x):
    raise NotImplementedError("write your pallas kernel here")


# speedup vs baseline: 5.7805x; 5.7805x over previous
import functools

import jax
import jax.numpy as jnp
from jax import lax
from jax.experimental import pallas as pl
from jax.experimental.pallas import tpu as pltpu
from jax.experimental.pallas import tpu_sc as plsc

_K = 512
_B = 64
_N = 8192
_L = 16
_NW = 32
_ROWS_PER_W = _B // _NW
_CHUNKS = _N // _L
_UNROLL = 8


def _lane_allreduce_sum(v):
    iota = lax.iota(jnp.int32, _L)
    dnums = lax.GatherDimensionNumbers(
        offset_dims=(), collapsed_slice_dims=(0,), start_index_map=(0,))
    for shift in (8, 4, 2, 1):
        idx = (iota + shift) & (_L - 1)
        rot = lax.gather(v, idx[:, None], dimension_numbers=dnums,
                         slice_sizes=(1,),
                         mode=lax.GatherScatterMode.PROMISE_IN_BOUNDS)
        v = v + rot
    return v


def _body(x_hbm, out_hbm, row_v, bits_v, out_v):
    wid = lax.axis_index("s") * 2 + lax.axis_index("c")
    ones = jnp.ones((_L,), jnp.int32)
    zeros = jnp.zeros((_L,), jnp.int32)
    for r in range(_ROWS_PER_W):
        row = wid * _ROWS_PER_W + r
        pltpu.sync_copy(x_hbm.at[row], row_v)

        def prep(i, carry):
            for u in range(_UNROLL):
                off = (i * _UNROLL + u) * _L
                v = row_v[pl.ds(off, _L)]
                bits_v[pl.ds(off, _L)] = lax.bitcast_convert_type(
                    jnp.abs(v), jnp.int32)
            return carry

        lax.fori_loop(0, _CHUNKS // _UNROLL, prep, jnp.int32(0))

        def bit_step(b, t):
            sh = jnp.broadcast_to(jnp.int32(30) - b, (_L,))
            cand = t | (ones << sh)

            def count_chunk(i, cnt):
                for u in range(_UNROLL):
                    off = (i * _UNROLL + u) * _L
                    ge = bits_v[pl.ds(off, _L)] >= cand
                    cnt = cnt + jnp.where(ge, ones, zeros)
                return cnt

            cnt = lax.fori_loop(0, _CHUNKS // _UNROLL, count_chunk, zeros)
            total = _lane_allreduce_sum(cnt)
            return jnp.where(total >= _K, cand, t)

        t = lax.fori_loop(0, 31, bit_step, zeros)

        def mask_chunk(i, carry):
            for u in range(_UNROLL):
                off = (i * _UNROLL + u) * _L
                v = row_v[pl.ds(off, _L)]
                keep = bits_v[pl.ds(off, _L)] >= carry
                out_v[pl.ds(off, _L)] = jnp.where(keep, v, jnp.float32(0))
            return carry

        lax.fori_loop(0, _CHUNKS // _UNROLL, mask_chunk, t)
        pltpu.sync_copy(out_v, out_hbm.at[row])


@jax.jit
def kernel(x):
    mesh = plsc.VectorSubcoreMesh(core_axis_name="c", subcore_axis_name="s")
    fn = functools.partial(
        pl.kernel,
        mesh=mesh,
        out_type=jax.ShapeDtypeStruct((_B, _N), jnp.float32),
        scratch_types=[
            pltpu.VMEM((_N,), jnp.float32),
            pltpu.VMEM((_N,), jnp.int32),
            pltpu.VMEM((_N,), jnp.float32),
        ],
    )(_body)
    return fn(x)
